# Initial kernel scaffold; baseline (speedup 1.0000x reference)
import jax
import jax.numpy as jnp
from jax.experimental import pallas as pl


def kernel(users, items, user_table, item_table, edge_user, edge_item):
    raise NotImplementedError("write your pallas kernel here")


# speedup vs baseline: 333.2268x; 333.2268x over previous
import functools

import jax
import jax.numpy as jnp
from jax import lax
from jax.experimental import pallas as pl
from jax.experimental.pallas import tpu as pltpu
from jax.experimental.pallas import tpu_sc as plsc

B = 16384
D = 64
NC = 2
NS = 16
NW = NC * NS
BPW = B // NW
L = 16
SCALE = 0.25


@functools.partial(
    pl.kernel,
    out_type=(
        jax.ShapeDtypeStruct((B, D), jnp.float32),
        jax.ShapeDtypeStruct((B, D), jnp.float32),
    ),
    mesh=plsc.VectorSubcoreMesh(core_axis_name="c", subcore_axis_name="s"),
    scratch_types=[
        pltpu.VMEM((BPW,), jnp.int32),
        pltpu.VMEM((BPW, D), jnp.float32),
        pltpu.SemaphoreType.DMA,
    ],
    compiler_params=pltpu.CompilerParams(use_tc_tiling_on_sc=False),
)
def _gather_scale(users_hbm, items_hbm, utab_hbm, itab_hbm,
                  out_u_hbm, out_i_hbm, idx_v, rows_v, sem):
    wid = lax.axis_index("s") * NC + lax.axis_index("c")
    base = wid * BPW

    def one_table(src_idx_hbm, tab_hbm, out_hbm):
        pltpu.sync_copy(src_idx_hbm.at[pl.ds(base, BPW)], idx_v)
        pltpu.async_copy(tab_hbm.at[idx_v], rows_v, sem).wait()

        def scale_row(i, _):
            for j in range(D // L):
                sl = pl.ds(j * L, L)
                rows_v[i, sl] = rows_v[i, sl] * SCALE
            return 0

        lax.fori_loop(0, BPW, scale_row, 0)
        pltpu.sync_copy(rows_v, out_hbm.at[pl.ds(base, BPW)])

    one_table(users_hbm, utab_hbm, out_u_hbm)
    one_table(items_hbm, itab_hbm, out_i_hbm)


def kernel(users, items, user_table, item_table, edge_user, edge_item):
    del edge_user, edge_item
    return _gather_scale(users, items, user_table, item_table)
